# Initial kernel scaffold; baseline (speedup 1.0000x reference)
import jax
import jax.numpy as jnp
from jax.experimental import pallas as pl


def kernel(input, adjacency_matrix, W, b):
    raise NotImplementedError("write your pallas kernel here")


# speedup vs baseline: 1.6595x; 1.6595x over previous
import jax
import jax.numpy as jnp
from jax.experimental import pallas as pl


def _sage_body(adj_ref, xe_ref, w_ref, b_ref, out_ref):
    in_f = w_ref.shape[1]
    mask = adj_ref[...].astype(jnp.bfloat16)
    accw = jnp.dot(mask, xe_ref[...], preferred_element_type=jnp.float32)
    acc = accw[:, :in_f]
    deg = accw[:, in_f:in_f + 1]
    agg = acc / deg
    out_ref[...] = jax.lax.dot_general(
        agg, w_ref[...], (((1,), (1,)), ((), ())),
        preferred_element_type=jnp.float32) + b_ref[...]


def kernel(input, adjacency_matrix, W, b):
    n, in_f = input.shape
    out_f = W.shape[0]
    bm = 400
    xe = jnp.concatenate(
        [input,
         jnp.ones((n, 1), input.dtype),
         jnp.zeros((n, 127), input.dtype)], axis=1).astype(jnp.bfloat16)
    b2 = b.reshape(1, out_f)
    return pl.pallas_call(
        _sage_body,
        out_shape=jax.ShapeDtypeStruct((n, out_f), jnp.float32),
        grid=(n // bm,),
        in_specs=[
            pl.BlockSpec((bm, n), lambda i: (i, 0)),
            pl.BlockSpec((n, in_f + 128), lambda i: (0, 0)),
            pl.BlockSpec((out_f, in_f), lambda i: (0, 0)),
            pl.BlockSpec((1, out_f), lambda i: (0, 0)),
        ],
        out_specs=pl.BlockSpec((bm, out_f), lambda i: (i, 0)),
    )(adjacency_matrix, xe, W, b2)
